# baseline (speedup 1.0000x reference)
import functools

import jax
import jax.numpy as jnp
from jax import lax
from jax.experimental import pallas as pl
from jax.experimental.pallas import tpu as pltpu
from jax.experimental.pallas import tpu_sc as plsc

N = 10000
E = 320000
HEADS = 8
BASES = 4
IN_FEATURES = 128
HIDDEN = 128
OUT_ROUNDED = 352
OUT_TRUE = 349

NC = 2
NS = 16
NW = NC * NS
CH = 128
NPAD = 10240
EPW = NPAD
NCH = EPW // CH
EP = NW * EPW
RPW = NPAD // NS

@functools.cache
def _mesh():
    return plsc.VectorSubcoreMesh(
        core_axis_name="c", subcore_axis_name="s", num_cores=NC, num_subcores=NS
    )


@functools.cache
def _make_deg():
    @functools.partial(
        pl.kernel,
        mesh=_mesh(),
        compiler_params=pltpu.CompilerParams(use_tc_tiling_on_sc=False),
        out_type=jax.ShapeDtypeStruct((NC, NPAD), jnp.float32),
        scratch_types=[
            pltpu.VMEM((NCH, CH), jnp.int32),
            pltpu.VMEM((CH,), jnp.float32),
            pltpu.VMEM_SHARED((NPAD,), jnp.float32),
        ],
    )
    def _deg_kernel(dst_hbm, zeros_hbm, out_hbm, didx, ones_v, deg_sp):
        c = lax.axis_index("c")
        s = lax.axis_index("s")
        w = s * NC + c
        for k in range(CH // 16):
            ones_v[pl.ds(k * 16, 16)] = jnp.ones((16,), jnp.float32)
        pltpu.sync_copy(zeros_hbm.at[pl.ds(s * RPW, RPW)],
                        deg_sp.at[pl.ds(s * RPW, RPW)])
        pltpu.sync_copy(dst_hbm.at[w], didx)
        plsc.subcore_barrier()

        @pl.loop(0, NCH)
        def _edge_chunk(j):
            pltpu.sync_copy(ones_v, deg_sp.at[didx.at[j]], add=True)

        plsc.subcore_barrier()
        pltpu.sync_copy(deg_sp.at[pl.ds(s * RPW, RPW)],
                        out_hbm.at[c, pl.ds(s * RPW, RPW)])

    return _deg_kernel


NBUF = 4


@functools.cache
def _make_agg(width):
    @functools.partial(
        pl.kernel,
        mesh=_mesh(),
        compiler_params=pltpu.CompilerParams(use_tc_tiling_on_sc=False),
        out_type=jax.ShapeDtypeStruct((NC, NPAD, width), jnp.float32),
        scratch_types=[
            pltpu.VMEM((NCH, CH), jnp.int32),
            pltpu.VMEM((NCH, CH), jnp.int32),
        ]
        + [pltpu.VMEM((CH, width), jnp.float32) for _ in range(NBUF)]
        + [pltpu.SemaphoreType.DMA for _ in range(2 * NBUF)]
        + [pltpu.VMEM_SHARED((NPAD, width), jnp.float32)],
    )
    def _agg_kernel(bases_hbm, src_hbm, dst_hbm, out_hbm, sidx, didx, *rest):
        rows = rest[:NBUF]
        gsem = rest[NBUF:2 * NBUF]
        ssem = rest[2 * NBUF:3 * NBUF]
        agg_sp = rest[3 * NBUF]
        c = lax.axis_index("c")
        s = lax.axis_index("s")
        w = s * NC + c
        pltpu.sync_copy(bases_hbm.at[pl.ds(s * RPW, RPW)],
                        agg_sp.at[pl.ds(s * RPW, RPW)])
        pltpu.sync_copy(src_hbm.at[w], sidx)
        pltpu.sync_copy(dst_hbm.at[w], didx)
        plsc.subcore_barrier()

        for b in range(NBUF - 1):
            pltpu.async_copy(bases_hbm.at[sidx.at[b]], rows[b], gsem[b])

        @pl.loop(0, NCH, step=NBUF)
        def _edge_chunk(j):
            for b in range(NBUF):
                jb = j + b
                pltpu.make_async_copy(bases_hbm.at[sidx.at[jb]], rows[b],
                                      gsem[b]).wait()
                pltpu.async_copy(rows[b], agg_sp.at[didx.at[jb]], ssem[b],
                                 add=True)
                bp = (b - 1) % NBUF

                @pl.when(jb >= 1)
                def _drain_prev():
                    pltpu.make_async_copy(rows[bp], agg_sp.at[didx.at[jb - 1]],
                                          ssem[bp]).wait()

                bn = (b + NBUF - 1) % NBUF

                @pl.when(jb + NBUF - 1 < NCH)
                def _prefetch():
                    pltpu.async_copy(bases_hbm.at[sidx.at[jb + NBUF - 1]],
                                     rows[bn], gsem[bn])

        bl = (NCH - 1) % NBUF
        pltpu.make_async_copy(rows[bl], agg_sp.at[didx.at[NCH - 1]],
                              ssem[bl]).wait()
        plsc.subcore_barrier()
        pltpu.sync_copy(agg_sp.at[pl.ds(s * RPW, RPW)],
                        out_hbm.at[c, pl.ds(s * RPW, RPW)])

    return _agg_kernel


NCH1 = EP // NS // CH
NBUF1 = 2


@functools.cache
def _make_agg_split(width):
    @functools.partial(
        pl.kernel,
        mesh=_mesh(),
        compiler_params=pltpu.CompilerParams(use_tc_tiling_on_sc=False),
        out_type=jax.ShapeDtypeStruct((NC, NPAD, width), jnp.float32),
        scratch_types=[
            pltpu.VMEM((NCH1, CH), jnp.int32),
            pltpu.VMEM((NCH1, CH), jnp.int32),
        ]
        + [pltpu.VMEM((CH, width), jnp.float32) for _ in range(NBUF1)]
        + [pltpu.SemaphoreType.DMA for _ in range(2 * NBUF1)]
        + [pltpu.VMEM_SHARED((NPAD, width), jnp.float32)],
    )
    def _agg_kernel(bases_hbm2, src_hbm, dst_hbm, out_hbm, sidx, didx, *rest):
        rows = rest[:NBUF1]
        gsem = rest[NBUF1:2 * NBUF1]
        ssem = rest[2 * NBUF1:3 * NBUF1]
        agg_sp = rest[3 * NBUF1]
        c = lax.axis_index("c")
        s = lax.axis_index("s")
        bases_hbm = bases_hbm2.at[c]
        pltpu.sync_copy(src_hbm.at[c, s], sidx)
        pltpu.sync_copy(dst_hbm.at[c, s], didx)
        pltpu.sync_copy(bases_hbm.at[pl.ds(s * RPW, RPW)],
                        agg_sp.at[pl.ds(s * RPW, RPW)])
        plsc.subcore_barrier()
        for b in range(NBUF1 - 1):
            pltpu.async_copy(bases_hbm.at[sidx.at[b]], rows[b], gsem[b])

        @pl.loop(0, NCH1, step=NBUF1)
        def _edge_chunk(j):
            for b in range(NBUF1):
                jb = j + b
                pltpu.make_async_copy(bases_hbm.at[sidx.at[jb]], rows[b],
                                      gsem[b]).wait()
                pltpu.async_copy(rows[b], agg_sp.at[didx.at[jb]], ssem[b],
                                 add=True)
                bp = (b - 1) % NBUF1

                @pl.when(jb >= 1)
                def _drain_prev():
                    pltpu.make_async_copy(rows[bp], agg_sp.at[didx.at[jb - 1]],
                                          ssem[bp]).wait()

                bn = (b + NBUF1 - 1) % NBUF1

                @pl.when(jb + NBUF1 - 1 < NCH1)
                def _prefetch():
                    pltpu.async_copy(bases_hbm.at[sidx.at[jb + NBUF1 - 1]],
                                     rows[bn], gsem[bn])

        bl = (NCH1 - 1) % NBUF1
        pltpu.make_async_copy(rows[bl], agg_sp.at[didx.at[NCH1 - 1]],
                              ssem[bl]).wait()
        plsc.subcore_barrier()
        pltpu.sync_copy(agg_sp.at[pl.ds(s * RPW, RPW)],
                        out_hbm.at[c, pl.ds(s * RPW, RPW)])

    return _agg_kernel


ROWS = 512
GRID = NPAD // ROWS


def _dinv(deg_blk):
    return lax.rsqrt(1.0 + deg_blk[:, 0:1] + deg_blk[:, 1:2])


def _combine(wgt, agg, width):
    chead = width // BASES
    cols = []
    for h in range(HEADS):
        acc = wgt[:, h * BASES:h * BASES + 1] * agg[:, 0:chead]
        for b in range(1, BASES):
            acc = acc + (wgt[:, h * BASES + b:h * BASES + b + 1]
                         * agg[:, b * chead:(b + 1) * chead])
        cols.append(acc)
    return jnp.concatenate(cols, axis=1)


def _prep_body(x_ref, deg_ref, wcat_ref, bc_ref, bases_ref, wgt_ref, *, wout):
    dinv = _dinv(deg_ref[...])
    y = jnp.dot(x_ref[...], wcat_ref[...], preferred_element_type=jnp.float32)
    bases_ref[...] = y[:, :wout] * dinv
    wgt_ref[...] = y[:, wout:] + bc_ref[...]


def _mid_body(agg0_ref, agg1_ref, bases_ref, wgt_ref, deg_ref, bias_ref,
              wcat_ref, bc_ref, *out_refs, wprev, wnext):
    dinv = _dinv(deg_ref[...])
    agg = (agg0_ref[...] + agg1_ref[...] - bases_ref[...]) * dinv
    h = _combine(wgt_ref[...], agg, wprev) + bias_ref[...]
    h = jnp.maximum(h, 0.0)
    y = jnp.dot(h, wcat_ref[...], preferred_element_type=jnp.float32)
    off = 0
    for ref, w in zip(out_refs[:-1], wnext):
        ref[...] = y[:, off:off + w] * dinv
        off += w
    out_refs[-1][...] = y[:, off:] + bc_ref[...]


def _final_body(agga_ref, aggb_ref, wgt_ref, deg_ref, bias_ref, out_ref):
    dinv = _dinv(deg_ref[...])
    agg = jnp.concatenate([agga_ref[...], aggb_ref[:, :80]], axis=1) * dinv
    h = _combine(wgt_ref[...], agg, 176) + bias_ref[...]
    col = lax.broadcasted_iota(jnp.int32, (ROWS, OUT_ROUNDED), 1)
    hm = jnp.where(col < OUT_TRUE, h, -jnp.inf)
    m = jnp.max(hm, axis=1, keepdims=True)
    ssum = jnp.sum(jnp.exp(hm - m), axis=1, keepdims=True)
    out_ref[...] = (h - m) - jnp.log(ssum)


def _row_spec(width):
    return pl.BlockSpec((ROWS, width), lambda i: (i, 0))


def _full_spec(r, c):
    return pl.BlockSpec((r, c), lambda i: (0, 0))


def _tc_prep(xp, degT, wcat, bc, wout):
    return pl.pallas_call(
        functools.partial(_prep_body, wout=wout),
        grid=(GRID,),
        in_specs=[
            _row_spec(IN_FEATURES),
            _row_spec(2),
            _full_spec(*wcat.shape),
            _full_spec(1, HEADS * BASES),
        ],
        out_specs=[_row_spec(wout), _row_spec(HEADS * BASES)],
        out_shape=[
            jax.ShapeDtypeStruct((NPAD, wout), jnp.float32),
            jax.ShapeDtypeStruct((NPAD, HEADS * BASES), jnp.float32),
        ],
    )(xp, degT, wcat, bc)


def _tc_mid(agg0, agg1, bases, wgt, degT, bias, wcat, bc, wprev, cout, wnext):
    return pl.pallas_call(
        functools.partial(_mid_body, wprev=wprev, wnext=wnext),
        grid=(GRID,),
        in_specs=[
            _row_spec(wprev),
            _row_spec(wprev),
            _row_spec(wprev),
            _row_spec(HEADS * BASES),
            _row_spec(2),
            _full_spec(1, cout),
            _full_spec(*wcat.shape),
            _full_spec(1, HEADS * BASES),
        ],
        out_specs=[_row_spec(w) for w in wnext] + [_row_spec(HEADS * BASES)],
        out_shape=[jax.ShapeDtypeStruct((NPAD, w), jnp.float32) for w in wnext]
        + [jax.ShapeDtypeStruct((NPAD, HEADS * BASES), jnp.float32)],
    )(agg0, agg1, bases, wgt, degT, bias, wcat, bc)


def _tc_final(agga, aggb, wgt, degT, bias):
    return pl.pallas_call(
        _final_body,
        grid=(GRID,),
        in_specs=[
            _row_spec(96),
            _row_spec(96),
            _row_spec(HEADS * BASES),
            _row_spec(2),
            _full_spec(1, OUT_ROUNDED),
        ],
        out_specs=[_row_spec(OUT_ROUNDED)],
        out_shape=[jax.ShapeDtypeStruct((NPAD, OUT_ROUNDED), jnp.float32)],
    )(agga, aggb, wgt, degT, bias)[0]


def kernel(x, edge_index, Wb0, Wc0, bc0, b0, Wb1, Wc1, bc1, b1, Wb2, Wc2, bc2, b2):
    f32 = jnp.float32
    pad = N + (jnp.arange(EP - E, dtype=jnp.int32) % (NPAD - N))
    srcR = jnp.concatenate([edge_index[0], pad]).reshape(NW, NCH, CH)
    dstR = jnp.concatenate([edge_index[1], pad]).reshape(NW, NCH, CH)
    zeros1 = jnp.zeros((NPAD,), f32)
    xp = jnp.pad(x, ((0, NPAD - N), (0, 0)))

    deg = _make_deg()(dstR, zeros1)
    degT = jnp.transpose(deg)

    wcat0 = jnp.concatenate([Wb0, Wc0], axis=1)
    wcat1 = jnp.concatenate([Wb1, Wc1], axis=1)
    wcat2 = jnp.concatenate(
        [Wb2, jnp.zeros((HIDDEN, 16), f32), Wc2], axis=1)

    bases0, wgt0 = _tc_prep(xp, degT, wcat0, bc0.reshape(1, -1), 64)
    ag = _make_agg(64)(bases0, srcR, dstR)
    bases1, wgt1 = _tc_mid(ag[0], ag[1], bases0, wgt0, degT, b0.reshape(1, -1),
                           wcat1, bc1.reshape(1, -1), 64, HIDDEN, (64,))
    ag = _make_agg(64)(bases1, srcR, dstR)
    bases2a, bases2b, wgt2 = _tc_mid(
        ag[0], ag[1], bases1, wgt1, degT, b1.reshape(1, -1),
        wcat2, bc2.reshape(1, -1), 64, HIDDEN, (96, 96))
    srcR1 = jnp.stack([srcR, srcR]).reshape(NC, NS, NCH1, CH)
    dstR1 = jnp.stack([dstR, dstR]).reshape(NC, NS, NCH1, CH)
    bases2 = jnp.stack([bases2a, bases2b])
    ag2 = _make_agg_split(96)(bases2, srcR1, dstR1)
    out = _tc_final(ag2[0], ag2[1], wgt2, degT, b2.reshape(1, -1))
    return out[:N, :OUT_TRUE]


# speedup vs baseline: 1.0765x; 1.0765x over previous
import functools

import jax
import jax.numpy as jnp
from jax import lax
from jax.experimental import pallas as pl
from jax.experimental.pallas import tpu as pltpu
from jax.experimental.pallas import tpu_sc as plsc

N = 10000
E = 320000
HEADS = 8
BASES = 4
IN_FEATURES = 128
HIDDEN = 128
OUT_ROUNDED = 352
OUT_TRUE = 349

NC = 2
NS = 16
NW = NC * NS
CH = 128
NPAD = 10240
EPW = NPAD
NCH = EPW // CH
EP = NW * EPW
RPW = NPAD // NS

@functools.cache
def _mesh():
    return plsc.VectorSubcoreMesh(
        core_axis_name="c", subcore_axis_name="s", num_cores=NC, num_subcores=NS
    )


@functools.cache
def _make_deg():
    @functools.partial(
        pl.kernel,
        mesh=_mesh(),
        compiler_params=pltpu.CompilerParams(use_tc_tiling_on_sc=False),
        out_type=jax.ShapeDtypeStruct((NC, NPAD), jnp.float32),
        scratch_types=[
            pltpu.VMEM((NCH, CH), jnp.int32),
            pltpu.VMEM((CH,), jnp.float32),
            pltpu.VMEM_SHARED((NPAD,), jnp.float32),
        ],
    )
    def _deg_kernel(dst_hbm, zeros_hbm, out_hbm, didx, ones_v, deg_sp):
        c = lax.axis_index("c")
        s = lax.axis_index("s")
        w = s * NC + c
        for k in range(CH // 16):
            ones_v[pl.ds(k * 16, 16)] = jnp.ones((16,), jnp.float32)
        pltpu.sync_copy(zeros_hbm.at[pl.ds(s * RPW, RPW)],
                        deg_sp.at[pl.ds(s * RPW, RPW)])
        pltpu.sync_copy(dst_hbm.at[w], didx)
        plsc.subcore_barrier()

        @pl.loop(0, NCH)
        def _edge_chunk(j):
            pltpu.sync_copy(ones_v, deg_sp.at[didx.at[j]], add=True)

        plsc.subcore_barrier()
        pltpu.sync_copy(deg_sp.at[pl.ds(s * RPW, RPW)],
                        out_hbm.at[c, pl.ds(s * RPW, RPW)])

    return _deg_kernel


NBUF = 4


@functools.cache
def _make_agg(width):
    @functools.partial(
        pl.kernel,
        mesh=_mesh(),
        compiler_params=pltpu.CompilerParams(use_tc_tiling_on_sc=False),
        out_type=jax.ShapeDtypeStruct((NC, NPAD, width), jnp.float32),
        scratch_types=[
            pltpu.VMEM((NCH, CH), jnp.int32),
            pltpu.VMEM((NCH, CH), jnp.int32),
        ]
        + [pltpu.VMEM((CH, width), jnp.float32) for _ in range(NBUF)]
        + [pltpu.SemaphoreType.DMA for _ in range(2 * NBUF)]
        + [pltpu.VMEM_SHARED((NPAD, width), jnp.float32)],
    )
    def _agg_kernel(bases_hbm, src_hbm, dst_hbm, out_hbm, sidx, didx, *rest):
        rows = rest[:NBUF]
        gsem = rest[NBUF:2 * NBUF]
        ssem = rest[2 * NBUF:3 * NBUF]
        agg_sp = rest[3 * NBUF]
        c = lax.axis_index("c")
        s = lax.axis_index("s")
        w = s * NC + c
        pltpu.sync_copy(bases_hbm.at[pl.ds(s * RPW, RPW)],
                        agg_sp.at[pl.ds(s * RPW, RPW)])
        pltpu.sync_copy(src_hbm.at[w], sidx)
        pltpu.sync_copy(dst_hbm.at[w], didx)
        plsc.subcore_barrier()

        for b in range(NBUF - 1):
            pltpu.async_copy(bases_hbm.at[sidx.at[b]], rows[b], gsem[b])

        @pl.loop(0, NCH, step=NBUF)
        def _edge_chunk(j):
            for b in range(NBUF):
                jb = j + b
                pltpu.make_async_copy(bases_hbm.at[sidx.at[jb]], rows[b],
                                      gsem[b]).wait()
                pltpu.async_copy(rows[b], agg_sp.at[didx.at[jb]], ssem[b],
                                 add=True)
                bp = (b - 1) % NBUF

                @pl.when(jb >= 1)
                def _drain_prev():
                    pltpu.make_async_copy(rows[bp], agg_sp.at[didx.at[jb - 1]],
                                          ssem[bp]).wait()

                bn = (b + NBUF - 1) % NBUF

                @pl.when(jb + NBUF - 1 < NCH)
                def _prefetch():
                    pltpu.async_copy(bases_hbm.at[sidx.at[jb + NBUF - 1]],
                                     rows[bn], gsem[bn])

        bl = (NCH - 1) % NBUF
        pltpu.make_async_copy(rows[bl], agg_sp.at[didx.at[NCH - 1]],
                              ssem[bl]).wait()
        plsc.subcore_barrier()
        pltpu.sync_copy(agg_sp.at[pl.ds(s * RPW, RPW)],
                        out_hbm.at[c, pl.ds(s * RPW, RPW)])

    return _agg_kernel


NCH1 = EP // NS // CH
BLK = 40


@functools.cache
def _make_agg_split(width):
    @functools.partial(
        pl.kernel,
        mesh=_mesh(),
        compiler_params=pltpu.CompilerParams(use_tc_tiling_on_sc=False),
        out_type=jax.ShapeDtypeStruct((NC, NPAD, width), jnp.float32),
        scratch_types=[
            pltpu.VMEM((2, BLK, CH), jnp.int32),
            pltpu.VMEM((2, BLK, CH), jnp.int32),
        ]
        + [pltpu.VMEM((CH, width), jnp.float32) for _ in range(NBUF)]
        + [pltpu.SemaphoreType.DMA for _ in range(2 * NBUF)]
        + [pltpu.SemaphoreType.DMA]
        + [pltpu.VMEM_SHARED((NPAD, width), jnp.float32)],
    )
    def _agg_kernel(bases_hbm2, src_hbm, dst_hbm, out_hbm, sidx, didx, *rest):
        rows = rest[:NBUF]
        gsem = rest[NBUF:2 * NBUF]
        ssem = rest[2 * NBUF:3 * NBUF]
        isem = rest[3 * NBUF]
        agg_sp = rest[3 * NBUF + 1]
        c = lax.axis_index("c")
        s = lax.axis_index("s")
        bases_hbm = bases_hbm2.at[c]

        def _sid(jb):
            return sidx.at[(jb // BLK) % 2, jb % BLK]

        def _did(jb):
            return didx.at[(jb // BLK) % 2, jb % BLK]

        pltpu.sync_copy(src_hbm.at[c, s, pl.ds(0, BLK)], sidx.at[0])
        pltpu.sync_copy(dst_hbm.at[c, s, pl.ds(0, BLK)], didx.at[0])
        pltpu.sync_copy(bases_hbm.at[pl.ds(s * RPW, RPW)],
                        agg_sp.at[pl.ds(s * RPW, RPW)])
        plsc.subcore_barrier()
        for b in range(NBUF - 1):
            pltpu.async_copy(bases_hbm.at[_sid(b)], rows[b], gsem[b])

        @pl.loop(0, NCH1, step=NBUF)
        def _edge_chunk(j):
            for b in range(NBUF):
                jb = j + b
                blk = jb // BLK
                off = jb - blk * BLK
                pltpu.make_async_copy(bases_hbm.at[_sid(jb)], rows[b],
                                      gsem[b]).wait()
                pltpu.async_copy(rows[b], agg_sp.at[_did(jb)], ssem[b],
                                 add=True)
                bp = (b - 1) % NBUF

                @pl.when(jb >= 1)
                def _drain_prev():
                    pltpu.make_async_copy(rows[bp], agg_sp.at[_did(jb - 1)],
                                          ssem[bp]).wait()

                @pl.when((off == 0) & (jb + BLK < NCH1))
                def _load_next_idx():
                    nb = (blk + 1) % 2
                    pltpu.async_copy(
                        src_hbm.at[c, s, pl.ds((blk + 1) * BLK, BLK)],
                        sidx.at[nb], isem)
                    pltpu.async_copy(
                        dst_hbm.at[c, s, pl.ds((blk + 1) * BLK, BLK)],
                        didx.at[nb], isem)

                @pl.when((off == BLK - NBUF) & (jb + NBUF < NCH1))
                def _wait_next_idx():
                    nb = (blk + 1) % 2
                    pltpu.make_async_copy(
                        src_hbm.at[c, s, pl.ds((blk + 1) * BLK, BLK)],
                        sidx.at[nb], isem).wait()
                    pltpu.make_async_copy(
                        dst_hbm.at[c, s, pl.ds((blk + 1) * BLK, BLK)],
                        didx.at[nb], isem).wait()

                bn = (b + NBUF - 1) % NBUF

                @pl.when(jb + NBUF - 1 < NCH1)
                def _prefetch():
                    pltpu.async_copy(bases_hbm.at[_sid(jb + NBUF - 1)],
                                     rows[bn], gsem[bn])

        bl = (NCH1 - 1) % NBUF
        pltpu.make_async_copy(rows[bl], agg_sp.at[_did(NCH1 - 1)],
                              ssem[bl]).wait()
        plsc.subcore_barrier()
        pltpu.sync_copy(agg_sp.at[pl.ds(s * RPW, RPW)],
                        out_hbm.at[c, pl.ds(s * RPW, RPW)])

    return _agg_kernel


ROWS = 512
GRID = NPAD // ROWS


def _dinv(deg_blk):
    return lax.rsqrt(1.0 + deg_blk[:, 0:1] + deg_blk[:, 1:2])


def _combine(wgt, agg, width):
    chead = width // BASES
    cols = []
    for h in range(HEADS):
        acc = wgt[:, h * BASES:h * BASES + 1] * agg[:, 0:chead]
        for b in range(1, BASES):
            acc = acc + (wgt[:, h * BASES + b:h * BASES + b + 1]
                         * agg[:, b * chead:(b + 1) * chead])
        cols.append(acc)
    return jnp.concatenate(cols, axis=1)


def _prep_body(x_ref, deg_ref, wcat_ref, bc_ref, bases_ref, wgt_ref, *, wout):
    dinv = _dinv(deg_ref[...])
    y = jnp.dot(x_ref[...], wcat_ref[...], preferred_element_type=jnp.float32)
    bases_ref[...] = y[:, :wout] * dinv
    wgt_ref[...] = y[:, wout:] + bc_ref[...]


def _mid_body(agg0_ref, agg1_ref, bases_ref, wgt_ref, deg_ref, bias_ref,
              wcat_ref, bc_ref, *out_refs, wprev, wnext):
    dinv = _dinv(deg_ref[...])
    agg = (agg0_ref[...] + agg1_ref[...] - bases_ref[...]) * dinv
    h = _combine(wgt_ref[...], agg, wprev) + bias_ref[...]
    h = jnp.maximum(h, 0.0)
    y = jnp.dot(h, wcat_ref[...], preferred_element_type=jnp.float32)
    off = 0
    for ref, w in zip(out_refs[:-1], wnext):
        ref[...] = y[:, off:off + w] * dinv
        off += w
    out_refs[-1][...] = y[:, off:] + bc_ref[...]


def _final_body(agga_ref, aggb_ref, wgt_ref, deg_ref, bias_ref, out_ref):
    dinv = _dinv(deg_ref[...])
    agg = jnp.concatenate([agga_ref[...], aggb_ref[:, :80]], axis=1) * dinv
    h = _combine(wgt_ref[...], agg, 176) + bias_ref[...]
    col = lax.broadcasted_iota(jnp.int32, (ROWS, OUT_ROUNDED), 1)
    hm = jnp.where(col < OUT_TRUE, h, -jnp.inf)
    m = jnp.max(hm, axis=1, keepdims=True)
    ssum = jnp.sum(jnp.exp(hm - m), axis=1, keepdims=True)
    out_ref[...] = (h - m) - jnp.log(ssum)


def _row_spec(width):
    return pl.BlockSpec((ROWS, width), lambda i: (i, 0))


def _full_spec(r, c):
    return pl.BlockSpec((r, c), lambda i: (0, 0))


def _tc_prep(xp, degT, wcat, bc, wout):
    return pl.pallas_call(
        functools.partial(_prep_body, wout=wout),
        grid=(GRID,),
        in_specs=[
            _row_spec(IN_FEATURES),
            _row_spec(2),
            _full_spec(*wcat.shape),
            _full_spec(1, HEADS * BASES),
        ],
        out_specs=[_row_spec(wout), _row_spec(HEADS * BASES)],
        out_shape=[
            jax.ShapeDtypeStruct((NPAD, wout), jnp.float32),
            jax.ShapeDtypeStruct((NPAD, HEADS * BASES), jnp.float32),
        ],
    )(xp, degT, wcat, bc)


def _tc_mid(agg0, agg1, bases, wgt, degT, bias, wcat, bc, wprev, cout, wnext):
    return pl.pallas_call(
        functools.partial(_mid_body, wprev=wprev, wnext=wnext),
        grid=(GRID,),
        in_specs=[
            _row_spec(wprev),
            _row_spec(wprev),
            _row_spec(wprev),
            _row_spec(HEADS * BASES),
            _row_spec(2),
            _full_spec(1, cout),
            _full_spec(*wcat.shape),
            _full_spec(1, HEADS * BASES),
        ],
        out_specs=[_row_spec(w) for w in wnext] + [_row_spec(HEADS * BASES)],
        out_shape=[jax.ShapeDtypeStruct((NPAD, w), jnp.float32) for w in wnext]
        + [jax.ShapeDtypeStruct((NPAD, HEADS * BASES), jnp.float32)],
    )(agg0, agg1, bases, wgt, degT, bias, wcat, bc)


def _tc_final(agga, aggb, wgt, degT, bias):
    return pl.pallas_call(
        _final_body,
        grid=(GRID,),
        in_specs=[
            _row_spec(96),
            _row_spec(96),
            _row_spec(HEADS * BASES),
            _row_spec(2),
            _full_spec(1, OUT_ROUNDED),
        ],
        out_specs=[_row_spec(OUT_ROUNDED)],
        out_shape=[jax.ShapeDtypeStruct((NPAD, OUT_ROUNDED), jnp.float32)],
    )(agga, aggb, wgt, degT, bias)[0]


def kernel(x, edge_index, Wb0, Wc0, bc0, b0, Wb1, Wc1, bc1, b1, Wb2, Wc2, bc2, b2):
    f32 = jnp.float32
    pad = N + (jnp.arange(EP - E, dtype=jnp.int32) % (NPAD - N))
    srcR = jnp.concatenate([edge_index[0], pad]).reshape(NW, NCH, CH)
    dstR = jnp.concatenate([edge_index[1], pad]).reshape(NW, NCH, CH)
    zeros1 = jnp.zeros((NPAD,), f32)
    xp = jnp.pad(x, ((0, NPAD - N), (0, 0)))

    deg = _make_deg()(dstR, zeros1)
    degT = jnp.transpose(deg)

    wcat0 = jnp.concatenate([Wb0, Wc0], axis=1)
    wcat1 = jnp.concatenate([Wb1, Wc1], axis=1)
    wcat2 = jnp.concatenate(
        [Wb2, jnp.zeros((HIDDEN, 16), f32), Wc2], axis=1)

    bases0, wgt0 = _tc_prep(xp, degT, wcat0, bc0.reshape(1, -1), 64)
    ag = _make_agg(64)(bases0, srcR, dstR)
    bases1, wgt1 = _tc_mid(ag[0], ag[1], bases0, wgt0, degT, b0.reshape(1, -1),
                           wcat1, bc1.reshape(1, -1), 64, HIDDEN, (64,))
    ag = _make_agg(64)(bases1, srcR, dstR)
    bases2a, bases2b, wgt2 = _tc_mid(
        ag[0], ag[1], bases1, wgt1, degT, b1.reshape(1, -1),
        wcat2, bc2.reshape(1, -1), 64, HIDDEN, (96, 96))
    srcR1 = jnp.stack([srcR, srcR]).reshape(NC, NS, NCH1, CH)
    dstR1 = jnp.stack([dstR, dstR]).reshape(NC, NS, NCH1, CH)
    bases2 = jnp.stack([bases2a, bases2b])
    ag2 = _make_agg_split(96)(bases2, srcR1, dstR1)
    out = _tc_final(ag2[0], ag2[1], wgt2, degT, b2.reshape(1, -1))
    return out[:N, :OUT_TRUE]


# speedup vs baseline: 1.0859x; 1.0088x over previous
import functools

import jax
import jax.numpy as jnp
from jax import lax
from jax.experimental import pallas as pl
from jax.experimental.pallas import tpu as pltpu
from jax.experimental.pallas import tpu_sc as plsc

N = 10000
E = 320000
HEADS = 8
BASES = 4
IN_FEATURES = 128
HIDDEN = 128
OUT_ROUNDED = 352
OUT_TRUE = 349

NC = 2
NS = 16
NW = NC * NS
CH = 128
NPAD = 10240
EPW = NPAD
NCH = EPW // CH
EP = NW * EPW
RPW = NPAD // NS

@functools.cache
def _mesh():
    return plsc.VectorSubcoreMesh(
        core_axis_name="c", subcore_axis_name="s", num_cores=NC, num_subcores=NS
    )


@functools.cache
def _make_deg():
    @functools.partial(
        pl.kernel,
        mesh=_mesh(),
        compiler_params=pltpu.CompilerParams(use_tc_tiling_on_sc=False),
        out_type=jax.ShapeDtypeStruct((NC, NPAD), jnp.float32),
        scratch_types=[
            pltpu.VMEM((NCH, CH), jnp.int32),
            pltpu.VMEM((CH,), jnp.float32),
            pltpu.VMEM_SHARED((NPAD,), jnp.float32),
        ],
    )
    def _deg_kernel(dst_hbm, zeros_hbm, out_hbm, didx, ones_v, deg_sp):
        c = lax.axis_index("c")
        s = lax.axis_index("s")
        w = s * NC + c
        for k in range(CH // 16):
            ones_v[pl.ds(k * 16, 16)] = jnp.ones((16,), jnp.float32)
        pltpu.sync_copy(zeros_hbm.at[pl.ds(s * RPW, RPW)],
                        deg_sp.at[pl.ds(s * RPW, RPW)])
        pltpu.sync_copy(dst_hbm.at[w], didx)
        plsc.subcore_barrier()

        @pl.loop(0, NCH)
        def _edge_chunk(j):
            pltpu.sync_copy(ones_v, deg_sp.at[didx.at[j]], add=True)

        plsc.subcore_barrier()
        pltpu.sync_copy(deg_sp.at[pl.ds(s * RPW, RPW)],
                        out_hbm.at[c, pl.ds(s * RPW, RPW)])

    return _deg_kernel


NBUF = 4


@functools.cache
def _make_agg(width):
    @functools.partial(
        pl.kernel,
        mesh=_mesh(),
        compiler_params=pltpu.CompilerParams(use_tc_tiling_on_sc=False),
        out_type=jax.ShapeDtypeStruct((NC, NPAD, width), jnp.float32),
        scratch_types=[
            pltpu.VMEM((NCH, CH), jnp.int32),
            pltpu.VMEM((NCH, CH), jnp.int32),
        ]
        + [pltpu.VMEM((CH, width), jnp.float32) for _ in range(NBUF)]
        + [pltpu.SemaphoreType.DMA for _ in range(2 * NBUF)]
        + [pltpu.VMEM_SHARED((NPAD, width), jnp.float32)],
    )
    def _agg_kernel(bases_hbm, src_hbm, dst_hbm, out_hbm, sidx, didx, *rest):
        rows = rest[:NBUF]
        gsem = rest[NBUF:2 * NBUF]
        ssem = rest[2 * NBUF:3 * NBUF]
        agg_sp = rest[3 * NBUF]
        c = lax.axis_index("c")
        s = lax.axis_index("s")
        w = s * NC + c
        pltpu.sync_copy(bases_hbm.at[pl.ds(s * RPW, RPW)],
                        agg_sp.at[pl.ds(s * RPW, RPW)])
        pltpu.sync_copy(src_hbm.at[w], sidx)
        pltpu.sync_copy(dst_hbm.at[w], didx)
        plsc.subcore_barrier()

        for b in range(NBUF - 1):
            pltpu.async_copy(bases_hbm.at[sidx.at[b]], rows[b], gsem[b])

        @pl.loop(0, NCH, step=NBUF)
        def _edge_chunk(j):
            for b in range(NBUF):
                jb = j + b
                pltpu.make_async_copy(bases_hbm.at[sidx.at[jb]], rows[b],
                                      gsem[b]).wait()
                pltpu.async_copy(rows[b], agg_sp.at[didx.at[jb]], ssem[b],
                                 add=True)
                bp = (b - 1) % NBUF

                @pl.when(jb >= 1)
                def _drain_prev():
                    pltpu.make_async_copy(rows[bp], agg_sp.at[didx.at[jb - 1]],
                                          ssem[bp]).wait()

                bn = (b + NBUF - 1) % NBUF

                @pl.when(jb + NBUF - 1 < NCH)
                def _prefetch():
                    pltpu.async_copy(bases_hbm.at[sidx.at[jb + NBUF - 1]],
                                     rows[bn], gsem[bn])

        bl = (NCH - 1) % NBUF
        pltpu.make_async_copy(rows[bl], agg_sp.at[didx.at[NCH - 1]],
                              ssem[bl]).wait()
        plsc.subcore_barrier()
        pltpu.sync_copy(agg_sp.at[pl.ds(s * RPW, RPW)],
                        out_hbm.at[c, pl.ds(s * RPW, RPW)])

    return _agg_kernel


NCH1 = EP // NS // CH
BLK = 40


@functools.cache
def _make_agg_split(width):
    @functools.partial(
        pl.kernel,
        mesh=_mesh(),
        compiler_params=pltpu.CompilerParams(use_tc_tiling_on_sc=False),
        out_type=jax.ShapeDtypeStruct((NC, NPAD, width), jnp.float32),
        scratch_types=[
            pltpu.VMEM((2, BLK, CH), jnp.int32),
            pltpu.VMEM((2, BLK, CH), jnp.int32),
        ]
        + [pltpu.VMEM((CH, width), jnp.float32) for _ in range(NBUF)]
        + [pltpu.SemaphoreType.DMA for _ in range(2 * NBUF)]
        + [pltpu.SemaphoreType.DMA]
        + [pltpu.VMEM_SHARED((NPAD, width), jnp.float32)],
    )
    def _agg_kernel(bases_hbm2, src_hbm, dst_hbm, out_hbm, sidx, didx, *rest):
        rows = rest[:NBUF]
        gsem = rest[NBUF:2 * NBUF]
        ssem = rest[2 * NBUF:3 * NBUF]
        isem = rest[3 * NBUF]
        agg_sp = rest[3 * NBUF + 1]
        c = lax.axis_index("c")
        s = lax.axis_index("s")
        bases_hbm = bases_hbm2.at[c]

        def _sid(jb):
            return sidx.at[(jb // BLK) % 2, jb % BLK]

        def _did(jb):
            return didx.at[(jb // BLK) % 2, jb % BLK]

        pltpu.sync_copy(src_hbm.at[c, s, pl.ds(0, BLK)], sidx.at[0])
        pltpu.sync_copy(dst_hbm.at[c, s, pl.ds(0, BLK)], didx.at[0])
        pltpu.sync_copy(bases_hbm.at[pl.ds(s * RPW, RPW)],
                        agg_sp.at[pl.ds(s * RPW, RPW)])
        plsc.subcore_barrier()
        for b in range(NBUF - 1):
            pltpu.async_copy(bases_hbm.at[_sid(b)], rows[b], gsem[b])

        @pl.loop(0, NCH1, step=NBUF)
        def _edge_chunk(j):
            for b in range(NBUF):
                jb = j + b
                blk = jb // BLK
                off = jb - blk * BLK
                pltpu.make_async_copy(bases_hbm.at[_sid(jb)], rows[b],
                                      gsem[b]).wait()
                pltpu.async_copy(rows[b], agg_sp.at[_did(jb)], ssem[b],
                                 add=True)
                bp = (b - 1) % NBUF

                @pl.when(jb >= 1)
                def _drain_prev():
                    pltpu.make_async_copy(rows[bp], agg_sp.at[_did(jb - 1)],
                                          ssem[bp]).wait()

                @pl.when((off == 0) & (jb + BLK < NCH1))
                def _load_next_idx():
                    nb = (blk + 1) % 2
                    pltpu.async_copy(
                        src_hbm.at[c, s, pl.ds((blk + 1) * BLK, BLK)],
                        sidx.at[nb], isem)
                    pltpu.async_copy(
                        dst_hbm.at[c, s, pl.ds((blk + 1) * BLK, BLK)],
                        didx.at[nb], isem)

                @pl.when((off == BLK - NBUF) & (jb + NBUF < NCH1))
                def _wait_next_idx():
                    nb = (blk + 1) % 2
                    pltpu.make_async_copy(
                        src_hbm.at[c, s, pl.ds((blk + 1) * BLK, BLK)],
                        sidx.at[nb], isem).wait()
                    pltpu.make_async_copy(
                        dst_hbm.at[c, s, pl.ds((blk + 1) * BLK, BLK)],
                        didx.at[nb], isem).wait()

                bn = (b + NBUF - 1) % NBUF

                @pl.when(jb + NBUF - 1 < NCH1)
                def _prefetch():
                    pltpu.async_copy(bases_hbm.at[_sid(jb + NBUF - 1)],
                                     rows[bn], gsem[bn])

        bl = (NCH1 - 1) % NBUF
        pltpu.make_async_copy(rows[bl], agg_sp.at[_did(NCH1 - 1)],
                              ssem[bl]).wait()
        plsc.subcore_barrier()
        pltpu.sync_copy(agg_sp.at[pl.ds(s * RPW, RPW)],
                        out_hbm.at[c, pl.ds(s * RPW, RPW)])

    return _agg_kernel


ROWS = 512
GRID = NPAD // ROWS


def _dinv(deg_blk):
    return lax.rsqrt(1.0 + deg_blk[:, 0:1] + deg_blk[:, 1:2])


def _combine(wgt, agg, width):
    chead = width // BASES
    cols = []
    for h in range(HEADS):
        acc = wgt[:, h * BASES:h * BASES + 1] * agg[:, 0:chead]
        for b in range(1, BASES):
            acc = acc + (wgt[:, h * BASES + b:h * BASES + b + 1]
                         * agg[:, b * chead:(b + 1) * chead])
        cols.append(acc)
    return jnp.concatenate(cols, axis=1)


def _prep_body(x_ref, deg_ref, wcat_ref, bc_ref, bases_ref, wgt_ref, *, wout):
    dinv = _dinv(deg_ref[...])
    y = jnp.dot(x_ref[...], wcat_ref[...], preferred_element_type=jnp.float32)
    bases_ref[...] = y[:, :wout] * dinv
    wgt_ref[...] = y[:, wout:] + bc_ref[...]


def _mid_body(agg0_ref, agg1_ref, bases_ref, wgt_ref, deg_ref, bias_ref,
              wcat_ref, bc_ref, *out_refs, wprev, wnext, stacked=False):
    dinv = _dinv(deg_ref[...])
    agg = (agg0_ref[...] + agg1_ref[...] - bases_ref[...]) * dinv
    h = _combine(wgt_ref[...], agg, wprev) + bias_ref[...]
    h = jnp.maximum(h, 0.0)
    y = jnp.dot(h, wcat_ref[...], preferred_element_type=jnp.float32)
    off = 0
    if stacked:
        w = wnext[0]
        for i in range(len(wnext)):
            out_refs[0][i] = y[:, off:off + w] * dinv
            off += w
        out_refs[-1][...] = y[:, off:] + bc_ref[...]
        return
    for ref, w in zip(out_refs[:-1], wnext):
        ref[...] = y[:, off:off + w] * dinv
        off += w
    out_refs[-1][...] = y[:, off:] + bc_ref[...]


def _final_body(agga_ref, aggb_ref, wgt_ref, deg_ref, bias_ref, out_ref):
    dinv = _dinv(deg_ref[...])
    agg = jnp.concatenate([agga_ref[...], aggb_ref[:, :80]], axis=1) * dinv
    h = _combine(wgt_ref[...], agg, 176) + bias_ref[...]
    col = lax.broadcasted_iota(jnp.int32, (ROWS, OUT_ROUNDED), 1)
    hm = jnp.where(col < OUT_TRUE, h, -jnp.inf)
    m = jnp.max(hm, axis=1, keepdims=True)
    ssum = jnp.sum(jnp.exp(hm - m), axis=1, keepdims=True)
    out_ref[...] = (h - m) - jnp.log(ssum)


def _row_spec(width):
    return pl.BlockSpec((ROWS, width), lambda i: (i, 0))


def _full_spec(r, c):
    return pl.BlockSpec((r, c), lambda i: (0, 0))


def _tc_prep(xp, degT, wcat, bc, wout):
    return pl.pallas_call(
        functools.partial(_prep_body, wout=wout),
        grid=(GRID,),
        in_specs=[
            _row_spec(IN_FEATURES),
            _row_spec(2),
            _full_spec(*wcat.shape),
            _full_spec(1, HEADS * BASES),
        ],
        out_specs=[_row_spec(wout), _row_spec(HEADS * BASES)],
        out_shape=[
            jax.ShapeDtypeStruct((NPAD, wout), jnp.float32),
            jax.ShapeDtypeStruct((NPAD, HEADS * BASES), jnp.float32),
        ],
    )(xp, degT, wcat, bc)


def _tc_mid(agg0, agg1, bases, wgt, degT, bias, wcat, bc, wprev, cout, wnext,
            stacked=False):
    if stacked:
        out_specs = [pl.BlockSpec((len(wnext), ROWS, wnext[0]),
                                  lambda i: (0, i, 0)),
                     _row_spec(HEADS * BASES)]
        out_shape = [jax.ShapeDtypeStruct((len(wnext), NPAD, wnext[0]),
                                          jnp.float32),
                     jax.ShapeDtypeStruct((NPAD, HEADS * BASES), jnp.float32)]
    else:
        out_specs = [_row_spec(w) for w in wnext] + [_row_spec(HEADS * BASES)]
        out_shape = ([jax.ShapeDtypeStruct((NPAD, w), jnp.float32)
                      for w in wnext]
                     + [jax.ShapeDtypeStruct((NPAD, HEADS * BASES),
                                             jnp.float32)])
    return pl.pallas_call(
        functools.partial(_mid_body, wprev=wprev, wnext=wnext,
                          stacked=stacked),
        grid=(GRID,),
        in_specs=[
            _row_spec(wprev),
            _row_spec(wprev),
            _row_spec(wprev),
            _row_spec(HEADS * BASES),
            _row_spec(2),
            _full_spec(1, cout),
            _full_spec(*wcat.shape),
            _full_spec(1, HEADS * BASES),
        ],
        out_specs=out_specs,
        out_shape=out_shape,
    )(agg0, agg1, bases, wgt, degT, bias, wcat, bc)


def _tc_final(agga, aggb, wgt, degT, bias):
    return pl.pallas_call(
        _final_body,
        grid=(GRID,),
        in_specs=[
            _row_spec(96),
            _row_spec(96),
            _row_spec(HEADS * BASES),
            _row_spec(2),
            _full_spec(1, OUT_ROUNDED),
        ],
        out_specs=[_row_spec(OUT_ROUNDED)],
        out_shape=[jax.ShapeDtypeStruct((NPAD, OUT_ROUNDED), jnp.float32)],
    )(agga, aggb, wgt, degT, bias)[0]


def kernel(x, edge_index, Wb0, Wc0, bc0, b0, Wb1, Wc1, bc1, b1, Wb2, Wc2, bc2, b2):
    f32 = jnp.float32
    pad = N + (jnp.arange(EP - E, dtype=jnp.int32) % (NPAD - N))
    srcR = jnp.concatenate([edge_index[0], pad]).reshape(NW, NCH, CH)
    dstR = jnp.concatenate([edge_index[1], pad]).reshape(NW, NCH, CH)
    zeros1 = jnp.zeros((NPAD,), f32)
    xp = jnp.pad(x, ((0, NPAD - N), (0, 0)))

    deg = _make_deg()(dstR, zeros1)
    degT = jnp.transpose(deg)

    wcat0 = jnp.concatenate([Wb0, Wc0], axis=1)
    wcat1 = jnp.concatenate([Wb1, Wc1], axis=1)
    wcat2 = jnp.concatenate(
        [Wb2, jnp.zeros((HIDDEN, 16), f32), Wc2], axis=1)

    bases0, wgt0 = _tc_prep(xp, degT, wcat0, bc0.reshape(1, -1), 64)
    ag = _make_agg(64)(bases0, srcR, dstR)
    bases1, wgt1 = _tc_mid(ag[0], ag[1], bases0, wgt0, degT, b0.reshape(1, -1),
                           wcat1, bc1.reshape(1, -1), 64, HIDDEN, (64,))
    ag = _make_agg(64)(bases1, srcR, dstR)
    bases2, wgt2 = _tc_mid(
        ag[0], ag[1], bases1, wgt1, degT, b1.reshape(1, -1),
        wcat2, bc2.reshape(1, -1), 64, HIDDEN, (96, 96), stacked=True)
    srcR1 = jnp.stack([srcR, srcR]).reshape(NC, NS, NCH1, CH)
    dstR1 = jnp.stack([dstR, dstR]).reshape(NC, NS, NCH1, CH)
    ag2 = _make_agg_split(96)(bases2, srcR1, dstR1)
    out = _tc_final(ag2[0], ag2[1], wgt2, degT, b2.reshape(1, -1))
    return out[:N, :OUT_TRUE]


# speedup vs baseline: 1.0873x; 1.0013x over previous
import functools

import jax
import jax.numpy as jnp
from jax import lax
from jax.experimental import pallas as pl
from jax.experimental.pallas import tpu as pltpu
from jax.experimental.pallas import tpu_sc as plsc

N = 10000
E = 320000
HEADS = 8
BASES = 4
IN_FEATURES = 128
HIDDEN = 128
OUT_ROUNDED = 352
OUT_TRUE = 349

NC = 2
NS = 16
NW = NC * NS
CH = 128
NPAD = 10240
EPW = NPAD
NCH = EPW // CH
EP = NW * EPW
RPW = NPAD // NS

@functools.cache
def _mesh():
    return plsc.VectorSubcoreMesh(
        core_axis_name="c", subcore_axis_name="s", num_cores=NC, num_subcores=NS
    )


@functools.cache
def _make_deg():
    @functools.partial(
        pl.kernel,
        mesh=_mesh(),
        compiler_params=pltpu.CompilerParams(use_tc_tiling_on_sc=False),
        out_type=jax.ShapeDtypeStruct((NC, NPAD), jnp.float32),
        scratch_types=[
            pltpu.VMEM((NCH, CH), jnp.int32),
            pltpu.VMEM((CH,), jnp.float32),
            pltpu.VMEM_SHARED((NPAD,), jnp.float32),
        ],
    )
    def _deg_kernel(dst_hbm, zeros_hbm, out_hbm, didx, ones_v, deg_sp):
        c = lax.axis_index("c")
        s = lax.axis_index("s")
        w = s * NC + c
        for k in range(CH // 16):
            ones_v[pl.ds(k * 16, 16)] = jnp.ones((16,), jnp.float32)
        pltpu.sync_copy(zeros_hbm.at[pl.ds(s * RPW, RPW)],
                        deg_sp.at[pl.ds(s * RPW, RPW)])
        pltpu.sync_copy(dst_hbm.at[w], didx)
        plsc.subcore_barrier()

        @pl.loop(0, NCH)
        def _edge_chunk(j):
            pltpu.sync_copy(ones_v, deg_sp.at[didx.at[j]], add=True)

        plsc.subcore_barrier()
        pltpu.sync_copy(deg_sp.at[pl.ds(s * RPW, RPW)],
                        out_hbm.at[c, pl.ds(s * RPW, RPW)])

    return _deg_kernel


NBUF = 4


@functools.cache
def _make_agg(width):
    @functools.partial(
        pl.kernel,
        mesh=_mesh(),
        compiler_params=pltpu.CompilerParams(use_tc_tiling_on_sc=False),
        out_type=jax.ShapeDtypeStruct((NC, NPAD, width), jnp.float32),
        scratch_types=[
            pltpu.VMEM((NCH, CH), jnp.int32),
            pltpu.VMEM((NCH, CH), jnp.int32),
        ]
        + [pltpu.VMEM((CH, width), jnp.float32) for _ in range(NBUF)]
        + [pltpu.SemaphoreType.DMA for _ in range(2 * NBUF)]
        + [pltpu.VMEM_SHARED((NPAD, width), jnp.float32)],
    )
    def _agg_kernel(bases_hbm, src_hbm, dst_hbm, out_hbm, sidx, didx, *rest):
        rows = rest[:NBUF]
        gsem = rest[NBUF:2 * NBUF]
        ssem = rest[2 * NBUF:3 * NBUF]
        agg_sp = rest[3 * NBUF]
        c = lax.axis_index("c")
        s = lax.axis_index("s")
        w = s * NC + c
        pltpu.sync_copy(bases_hbm.at[pl.ds(s * RPW, RPW)],
                        agg_sp.at[pl.ds(s * RPW, RPW)])
        pltpu.sync_copy(src_hbm.at[w], sidx)
        pltpu.sync_copy(dst_hbm.at[w], didx)
        plsc.subcore_barrier()

        for b in range(NBUF - 1):
            pltpu.async_copy(bases_hbm.at[sidx.at[b]], rows[b], gsem[b])

        @pl.loop(0, NCH, step=NBUF)
        def _edge_chunk(j):
            for b in range(NBUF):
                jb = j + b
                pltpu.make_async_copy(bases_hbm.at[sidx.at[jb]], rows[b],
                                      gsem[b]).wait()
                pltpu.async_copy(rows[b], agg_sp.at[didx.at[jb]], ssem[b],
                                 add=True)
                bp = (b - 1) % NBUF

                @pl.when(jb >= 1)
                def _drain_prev():
                    pltpu.make_async_copy(rows[bp], agg_sp.at[didx.at[jb - 1]],
                                          ssem[bp]).wait()

                bn = (b + NBUF - 1) % NBUF

                @pl.when(jb + NBUF - 1 < NCH)
                def _prefetch():
                    pltpu.async_copy(bases_hbm.at[sidx.at[jb + NBUF - 1]],
                                     rows[bn], gsem[bn])

        bl = (NCH - 1) % NBUF
        pltpu.make_async_copy(rows[bl], agg_sp.at[didx.at[NCH - 1]],
                              ssem[bl]).wait()
        plsc.subcore_barrier()
        pltpu.sync_copy(agg_sp.at[pl.ds(s * RPW, RPW)],
                        out_hbm.at[c, pl.ds(s * RPW, RPW)])

    return _agg_kernel


NCH1 = EP // NS // CH
BLK = 40


@functools.cache
def _make_agg_split(width):
    @functools.partial(
        pl.kernel,
        mesh=_mesh(),
        compiler_params=pltpu.CompilerParams(use_tc_tiling_on_sc=False),
        out_type=jax.ShapeDtypeStruct((NC, NPAD, width), jnp.float32),
        scratch_types=[
            pltpu.VMEM((2, BLK, CH), jnp.int32),
            pltpu.VMEM((2, BLK, CH), jnp.int32),
        ]
        + [pltpu.VMEM((CH, width), jnp.float32) for _ in range(NBUF)]
        + [pltpu.SemaphoreType.DMA for _ in range(2 * NBUF)]
        + [pltpu.SemaphoreType.DMA]
        + [pltpu.VMEM_SHARED((NPAD, width), jnp.float32)],
    )
    def _agg_kernel(bases_hbm2, src_hbm, dst_hbm, out_hbm, sidx, didx, *rest):
        rows = rest[:NBUF]
        gsem = rest[NBUF:2 * NBUF]
        ssem = rest[2 * NBUF:3 * NBUF]
        isem = rest[3 * NBUF]
        agg_sp = rest[3 * NBUF + 1]
        c = lax.axis_index("c")
        s = lax.axis_index("s")
        bases_hbm = bases_hbm2.at[c]

        def _sid(jb):
            return sidx.at[(jb // BLK) % 2, jb % BLK]

        def _did(jb):
            return didx.at[(jb // BLK) % 2, jb % BLK]

        pltpu.sync_copy(src_hbm.at[s, pl.ds(0, BLK)], sidx.at[0])
        pltpu.sync_copy(dst_hbm.at[s, pl.ds(0, BLK)], didx.at[0])
        pltpu.sync_copy(bases_hbm.at[pl.ds(s * RPW, RPW)],
                        agg_sp.at[pl.ds(s * RPW, RPW)])
        plsc.subcore_barrier()
        for b in range(NBUF - 1):
            pltpu.async_copy(bases_hbm.at[_sid(b)], rows[b], gsem[b])

        @pl.loop(0, NCH1, step=NBUF)
        def _edge_chunk(j):
            for b in range(NBUF):
                jb = j + b
                blk = jb // BLK
                off = jb - blk * BLK
                pltpu.make_async_copy(bases_hbm.at[_sid(jb)], rows[b],
                                      gsem[b]).wait()
                pltpu.async_copy(rows[b], agg_sp.at[_did(jb)], ssem[b],
                                 add=True)
                bp = (b - 1) % NBUF

                @pl.when(jb >= 1)
                def _drain_prev():
                    pltpu.make_async_copy(rows[bp], agg_sp.at[_did(jb - 1)],
                                          ssem[bp]).wait()

                @pl.when((off == 0) & (jb + BLK < NCH1))
                def _load_next_idx():
                    nb = (blk + 1) % 2
                    pltpu.async_copy(
                        src_hbm.at[s, pl.ds((blk + 1) * BLK, BLK)],
                        sidx.at[nb], isem)
                    pltpu.async_copy(
                        dst_hbm.at[s, pl.ds((blk + 1) * BLK, BLK)],
                        didx.at[nb], isem)

                @pl.when((off == BLK - NBUF) & (jb + NBUF < NCH1))
                def _wait_next_idx():
                    nb = (blk + 1) % 2
                    pltpu.make_async_copy(
                        src_hbm.at[s, pl.ds((blk + 1) * BLK, BLK)],
                        sidx.at[nb], isem).wait()
                    pltpu.make_async_copy(
                        dst_hbm.at[s, pl.ds((blk + 1) * BLK, BLK)],
                        didx.at[nb], isem).wait()

                bn = (b + NBUF - 1) % NBUF

                @pl.when(jb + NBUF - 1 < NCH1)
                def _prefetch():
                    pltpu.async_copy(bases_hbm.at[_sid(jb + NBUF - 1)],
                                     rows[bn], gsem[bn])

        bl = (NCH1 - 1) % NBUF
        pltpu.make_async_copy(rows[bl], agg_sp.at[_did(NCH1 - 1)],
                              ssem[bl]).wait()
        plsc.subcore_barrier()
        pltpu.sync_copy(agg_sp.at[pl.ds(s * RPW, RPW)],
                        out_hbm.at[c, pl.ds(s * RPW, RPW)])

    return _agg_kernel


ROWS = 512
GRID = NPAD // ROWS


def _dinv(deg_blk):
    return lax.rsqrt(1.0 + deg_blk[:, 0:1] + deg_blk[:, 1:2])


def _combine(wgt, agg, width):
    chead = width // BASES
    cols = []
    for h in range(HEADS):
        acc = wgt[:, h * BASES:h * BASES + 1] * agg[:, 0:chead]
        for b in range(1, BASES):
            acc = acc + (wgt[:, h * BASES + b:h * BASES + b + 1]
                         * agg[:, b * chead:(b + 1) * chead])
        cols.append(acc)
    return jnp.concatenate(cols, axis=1)


def _prep_body(x_ref, deg_ref, wcat_ref, bc_ref, bases_ref, wgt_ref, *, wout):
    dinv = _dinv(deg_ref[...])
    y = jnp.dot(x_ref[...], wcat_ref[...], preferred_element_type=jnp.float32)
    bases_ref[...] = y[:, :wout] * dinv
    wgt_ref[...] = y[:, wout:] + bc_ref[...]


def _mid_body(agg0_ref, agg1_ref, bases_ref, wgt_ref, deg_ref, bias_ref,
              wcat_ref, bc_ref, *out_refs, wprev, wnext, stacked=False):
    dinv = _dinv(deg_ref[...])
    agg = (agg0_ref[...] + agg1_ref[...] - bases_ref[...]) * dinv
    h = _combine(wgt_ref[...], agg, wprev) + bias_ref[...]
    h = jnp.maximum(h, 0.0)
    y = jnp.dot(h, wcat_ref[...], preferred_element_type=jnp.float32)
    off = 0
    if stacked:
        w = wnext[0]
        for i in range(len(wnext)):
            out_refs[0][i] = y[:, off:off + w] * dinv
            off += w
        out_refs[-1][...] = y[:, off:] + bc_ref[...]
        return
    for ref, w in zip(out_refs[:-1], wnext):
        ref[...] = y[:, off:off + w] * dinv
        off += w
    out_refs[-1][...] = y[:, off:] + bc_ref[...]


def _final_body(agga_ref, aggb_ref, wgt_ref, deg_ref, bias_ref, out_ref):
    dinv = _dinv(deg_ref[...])
    agg = jnp.concatenate([agga_ref[...], aggb_ref[:, :80]], axis=1) * dinv
    h = _combine(wgt_ref[...], agg, 176) + bias_ref[...]
    col = lax.broadcasted_iota(jnp.int32, (ROWS, OUT_ROUNDED), 1)
    hm = jnp.where(col < OUT_TRUE, h, -jnp.inf)
    m = jnp.max(hm, axis=1, keepdims=True)
    ssum = jnp.sum(jnp.exp(hm - m), axis=1, keepdims=True)
    out_ref[...] = (h - m) - jnp.log(ssum)


def _row_spec(width):
    return pl.BlockSpec((ROWS, width), lambda i: (i, 0))


def _full_spec(r, c):
    return pl.BlockSpec((r, c), lambda i: (0, 0))


def _tc_prep(xp, degT, wcat, bc, wout):
    return pl.pallas_call(
        functools.partial(_prep_body, wout=wout),
        grid=(GRID,),
        in_specs=[
            _row_spec(IN_FEATURES),
            _row_spec(2),
            _full_spec(*wcat.shape),
            _full_spec(1, HEADS * BASES),
        ],
        out_specs=[_row_spec(wout), _row_spec(HEADS * BASES)],
        out_shape=[
            jax.ShapeDtypeStruct((NPAD, wout), jnp.float32),
            jax.ShapeDtypeStruct((NPAD, HEADS * BASES), jnp.float32),
        ],
    )(xp, degT, wcat, bc)


def _tc_mid(agg0, agg1, bases, wgt, degT, bias, wcat, bc, wprev, cout, wnext,
            stacked=False):
    if stacked:
        out_specs = [pl.BlockSpec((len(wnext), ROWS, wnext[0]),
                                  lambda i: (0, i, 0)),
                     _row_spec(HEADS * BASES)]
        out_shape = [jax.ShapeDtypeStruct((len(wnext), NPAD, wnext[0]),
                                          jnp.float32),
                     jax.ShapeDtypeStruct((NPAD, HEADS * BASES), jnp.float32)]
    else:
        out_specs = [_row_spec(w) for w in wnext] + [_row_spec(HEADS * BASES)]
        out_shape = ([jax.ShapeDtypeStruct((NPAD, w), jnp.float32)
                      for w in wnext]
                     + [jax.ShapeDtypeStruct((NPAD, HEADS * BASES),
                                             jnp.float32)])
    return pl.pallas_call(
        functools.partial(_mid_body, wprev=wprev, wnext=wnext,
                          stacked=stacked),
        grid=(GRID,),
        in_specs=[
            _row_spec(wprev),
            _row_spec(wprev),
            _row_spec(wprev),
            _row_spec(HEADS * BASES),
            _row_spec(2),
            _full_spec(1, cout),
            _full_spec(*wcat.shape),
            _full_spec(1, HEADS * BASES),
        ],
        out_specs=out_specs,
        out_shape=out_shape,
    )(agg0, agg1, bases, wgt, degT, bias, wcat, bc)


def _tc_final(agga, aggb, wgt, degT, bias):
    return pl.pallas_call(
        _final_body,
        grid=(GRID,),
        in_specs=[
            _row_spec(96),
            _row_spec(96),
            _row_spec(HEADS * BASES),
            _row_spec(2),
            _full_spec(1, OUT_ROUNDED),
        ],
        out_specs=[_row_spec(OUT_ROUNDED)],
        out_shape=[jax.ShapeDtypeStruct((NPAD, OUT_ROUNDED), jnp.float32)],
    )(agga, aggb, wgt, degT, bias)[0]


def kernel(x, edge_index, Wb0, Wc0, bc0, b0, Wb1, Wc1, bc1, b1, Wb2, Wc2, bc2, b2):
    f32 = jnp.float32
    pad = N + (jnp.arange(EP - E, dtype=jnp.int32) % (NPAD - N))
    srcR = jnp.concatenate([edge_index[0], pad]).reshape(NW, NCH, CH)
    dstR = jnp.concatenate([edge_index[1], pad]).reshape(NW, NCH, CH)
    zeros1 = jnp.zeros((NPAD,), f32)
    xp = jnp.pad(x, ((0, NPAD - N), (0, 0)))

    deg = _make_deg()(dstR, zeros1)
    degT = jnp.transpose(deg)

    wcat0 = jnp.concatenate([Wb0, Wc0], axis=1)
    wcat1 = jnp.concatenate([Wb1, Wc1], axis=1)
    wcat2 = jnp.concatenate(
        [Wb2, jnp.zeros((HIDDEN, 16), f32), Wc2], axis=1)

    bases0, wgt0 = _tc_prep(xp, degT, wcat0, bc0.reshape(1, -1), 64)
    ag = _make_agg(64)(bases0, srcR, dstR)
    bases1, wgt1 = _tc_mid(ag[0], ag[1], bases0, wgt0, degT, b0.reshape(1, -1),
                           wcat1, bc1.reshape(1, -1), 64, HIDDEN, (64,))
    ag = _make_agg(64)(bases1, srcR, dstR)
    bases2, wgt2 = _tc_mid(
        ag[0], ag[1], bases1, wgt1, degT, b1.reshape(1, -1),
        wcat2, bc2.reshape(1, -1), 64, HIDDEN, (96, 96), stacked=True)
    srcR1 = srcR.reshape(NS, NCH1, CH)
    dstR1 = dstR.reshape(NS, NCH1, CH)
    ag2 = _make_agg_split(96)(bases2, srcR1, dstR1)
    out = _tc_final(ag2[0], ag2[1], wgt2, degT, b2.reshape(1, -1))
    return out[:N, :OUT_TRUE]
